# baseline (speedup 1.0000x reference)
import functools

import jax
import jax.numpy as jnp
from jax import lax
from jax.experimental import pallas as pl
from jax.experimental.pallas import tpu as pltpu
from jax.experimental.pallas import tpu_sc as plsc

MAX = 2000
HALF = 128
B, L = 4096, 50
NC, NS = 2, 16
NW = NC * NS
BB = B // NW
NBUF = 4
STRIPE = 128

_mesh = plsc.VectorSubcoreMesh(core_axis_name="c", subcore_axis_name="s")


@functools.partial(
    pl.kernel,
    out_type=jax.ShapeDtypeStruct((B, L, 2 * HALF), jnp.float32),
    mesh=_mesh,
    compiler_params=pltpu.CompilerParams(use_tc_tiling_on_sc=True),
    scratch_types=[
        pltpu.VMEM_SHARED((MAX, HALF), jnp.float32),
        pltpu.VMEM_SHARED((MAX, HALF), jnp.float32),
        pltpu.VMEM((BB, L), jnp.int32),
        pltpu.VMEM((BB, L), jnp.int32),
    ] + [pltpu.VMEM((L, HALF), jnp.float32)] * (2 * NBUF)
      + [pltpu.SemaphoreType.DMA] * (2 * NBUF),
)
def _emb_lookup(x_hbm, y_hbm, col_hbm, row_hbm, out_hbm,
                colsh, rowsh, xi, yi, *bufsems):
    bufs = bufsems[:2 * NBUF]
    sems = bufsems[2 * NBUF:]
    slots = [(bufs[2 * p], bufs[2 * p + 1], sems[2 * p], sems[2 * p + 1])
             for p in range(NBUF)]

    sid = lax.axis_index("s")
    wid = sid * NC + lax.axis_index("c")
    base_b = wid * BB

    row0 = sid * STRIPE

    @pl.when(sid < NS - 1)
    def _stage_full():
        pltpu.sync_copy(col_hbm.at[pl.ds(row0, STRIPE)],
                        colsh.at[pl.ds(row0, STRIPE)])
        pltpu.sync_copy(row_hbm.at[pl.ds(row0, STRIPE)],
                        rowsh.at[pl.ds(row0, STRIPE)])

    @pl.when(sid == NS - 1)
    def _stage_tail():
        tail = MAX - (NS - 1) * STRIPE
        t0 = (NS - 1) * STRIPE
        pltpu.sync_copy(col_hbm.at[pl.ds(t0, tail)],
                        colsh.at[pl.ds(t0, tail)])
        pltpu.sync_copy(row_hbm.at[pl.ds(t0, tail)],
                        rowsh.at[pl.ds(t0, tail)])

    pltpu.sync_copy(x_hbm.at[pl.ds(base_b, BB)], xi)
    pltpu.sync_copy(y_hbm.at[pl.ds(base_b, BB)], yi)
    plsc.subcore_barrier()

    def gather(k, xb, yb, gsem):
        pltpu.async_copy(colsh.at[xi.at[k]], xb, gsem)
        pltpu.async_copy(rowsh.at[yi.at[k]], yb, gsem)
        pltpu.make_async_copy(
            xb, out_hbm.at[0, pl.ds(0, L), pl.ds(0, HALF)], gsem).wait()
        pltpu.make_async_copy(
            yb, out_hbm.at[0, pl.ds(0, L), pl.ds(HALF, HALF)], gsem).wait()

    def issue_write(k, xb, yb, wsem):
        bb = base_b + k
        pltpu.async_copy(
            xb, out_hbm.at[bb, pl.ds(0, L), pl.ds(0, HALF)], wsem)
        pltpu.async_copy(
            yb, out_hbm.at[bb, pl.ds(0, L), pl.ds(HALF, HALF)], wsem)

    def drain_write(xb, yb, wsem):
        pltpu.make_async_copy(
            xb, out_hbm.at[0, pl.ds(0, L), pl.ds(0, HALF)], wsem).wait()
        pltpu.make_async_copy(
            yb, out_hbm.at[0, pl.ds(0, L), pl.ds(HALF, HALF)], wsem).wait()

    for p, (xb, yb, gsem, wsem) in enumerate(slots):
        gather(p, xb, yb, gsem)
        issue_write(p, xb, yb, wsem)

    def ring_body(g, carry):
        k0 = NBUF * g
        for p, (xb, yb, gsem, wsem) in enumerate(slots):
            k = k0 + p
            drain_write(xb, yb, wsem)
            gather(k, xb, yb, gsem)
            issue_write(k, xb, yb, wsem)
        return carry

    lax.fori_loop(1, BB // NBUF, ring_body, 0)

    for p, (xb, yb, gsem, wsem) in enumerate(slots):
        drain_write(xb, yb, wsem)


def kernel(x, y, col_weight, row_weight):
    out = _emb_lookup(x.astype(jnp.int32), y.astype(jnp.int32),
                      col_weight, row_weight)
    return out.reshape(1, B, L, 2 * HALF)


# speedup vs baseline: 1.0365x; 1.0365x over previous
import functools

import jax
import jax.numpy as jnp
from jax import lax
from jax.experimental import pallas as pl
from jax.experimental.pallas import tpu as pltpu
from jax.experimental.pallas import tpu_sc as plsc

MAX = 2000
HALF = 128
B, L = 4096, 50
NC, NS = 2, 16
NW = NC * NS
BB = B // NW
NBUF = 4
LAG = 2
STRIPE = 128

_mesh = plsc.VectorSubcoreMesh(core_axis_name="c", subcore_axis_name="s")


@functools.partial(
    pl.kernel,
    out_type=jax.ShapeDtypeStruct((B, L, 2 * HALF), jnp.float32),
    mesh=_mesh,
    compiler_params=pltpu.CompilerParams(use_tc_tiling_on_sc=True),
    scratch_types=[
        pltpu.VMEM_SHARED((MAX, HALF), jnp.float32),
        pltpu.VMEM_SHARED((MAX, HALF), jnp.float32),
        pltpu.VMEM((BB, L), jnp.int32),
        pltpu.VMEM((BB, L), jnp.int32),
    ] + [pltpu.VMEM((L, HALF), jnp.float32)] * (2 * NBUF)
      + [pltpu.SemaphoreType.DMA] * (2 * NBUF),
)
def _emb_lookup(x_hbm, y_hbm, col_hbm, row_hbm, out_hbm,
                colsh, rowsh, xi, yi, *bufsems):
    bufs = bufsems[:2 * NBUF]
    sems = bufsems[2 * NBUF:]
    slots = [(bufs[2 * p], bufs[2 * p + 1], sems[2 * p], sems[2 * p + 1])
             for p in range(NBUF)]

    sid = lax.axis_index("s")
    wid = sid * NC + lax.axis_index("c")
    base_b = wid * BB

    row0 = sid * STRIPE

    @pl.when(sid < NS - 1)
    def _stage_full():
        pltpu.sync_copy(col_hbm.at[pl.ds(row0, STRIPE)],
                        colsh.at[pl.ds(row0, STRIPE)])
        pltpu.sync_copy(row_hbm.at[pl.ds(row0, STRIPE)],
                        rowsh.at[pl.ds(row0, STRIPE)])

    @pl.when(sid == NS - 1)
    def _stage_tail():
        tail = MAX - (NS - 1) * STRIPE
        t0 = (NS - 1) * STRIPE
        pltpu.sync_copy(col_hbm.at[pl.ds(t0, tail)],
                        colsh.at[pl.ds(t0, tail)])
        pltpu.sync_copy(row_hbm.at[pl.ds(t0, tail)],
                        rowsh.at[pl.ds(t0, tail)])

    pltpu.sync_copy(x_hbm.at[pl.ds(base_b, BB)], xi)
    pltpu.sync_copy(y_hbm.at[pl.ds(base_b, BB)], yi)
    plsc.subcore_barrier()

    def issue_gather(k, p):
        xb, yb, gsem, _ = slots[p]
        pltpu.async_copy(colsh.at[xi.at[k]], xb, gsem)
        pltpu.async_copy(rowsh.at[yi.at[k]], yb, gsem)

    def drain_gather(p):
        xb, yb, gsem, _ = slots[p]
        pltpu.make_async_copy(
            xb, out_hbm.at[0, pl.ds(0, L), pl.ds(0, HALF)], gsem).wait()
        pltpu.make_async_copy(
            yb, out_hbm.at[0, pl.ds(0, L), pl.ds(HALF, HALF)], gsem).wait()

    def issue_write(k, p):
        xb, yb, _, wsem = slots[p]
        bb = base_b + k
        pltpu.async_copy(
            xb, out_hbm.at[bb, pl.ds(0, L), pl.ds(0, HALF)], wsem)
        pltpu.async_copy(
            yb, out_hbm.at[bb, pl.ds(0, L), pl.ds(HALF, HALF)], wsem)

    def drain_write(p):
        xb, yb, _, wsem = slots[p]
        pltpu.make_async_copy(
            xb, out_hbm.at[0, pl.ds(0, L), pl.ds(0, HALF)], wsem).wait()
        pltpu.make_async_copy(
            yb, out_hbm.at[0, pl.ds(0, L), pl.ds(HALF, HALF)], wsem).wait()


    for k in range(LAG):
        issue_gather(k, k % NBUF)

    for k in range(NBUF):
        p = k % NBUF
        drain_gather(p)
        issue_write(k, p)
        if k >= LAG:
            drain_write((k - LAG) % NBUF)
        issue_gather(k + LAG, (k + LAG) % NBUF)

    def ring_body(g, carry):
        k0 = NBUF * g
        for p in range(NBUF):
            k = k0 + p
            drain_gather(p)
            issue_write(k, p)
            drain_write((p + NBUF - LAG) % NBUF)
            issue_gather(k + LAG, (p + LAG) % NBUF)
        return carry

    lax.fori_loop(1, BB // NBUF - 1, ring_body, 0)

    for j in range(NBUF):
        k = BB - NBUF + j
        p = k % NBUF
        drain_gather(p)
        issue_write(k, p)
        drain_write((p + NBUF - LAG) % NBUF)
        if k + LAG < BB:
            issue_gather(k + LAG, (k + LAG) % NBUF)

    for k in range(BB - LAG, BB):
        drain_write(k % NBUF)


def kernel(x, y, col_weight, row_weight):
    out = _emb_lookup(x.astype(jnp.int32), y.astype(jnp.int32),
                      col_weight, row_weight)
    return out.reshape(1, B, L, 2 * HALF)


# speedup vs baseline: 1.0398x; 1.0032x over previous
import functools

import jax
import jax.numpy as jnp
from jax import lax
from jax.experimental import pallas as pl
from jax.experimental.pallas import tpu as pltpu
from jax.experimental.pallas import tpu_sc as plsc

MAX = 2000
HALF = 128
B, L = 4096, 50
NC, NS = 2, 16
NW = NC * NS
BB = B // NW
CH = 1
CHL = CH * L
NCK = BB // CH
NBUF = 4
STRIPE = 128

_mesh = plsc.VectorSubcoreMesh(core_axis_name="c", subcore_axis_name="s")


@functools.partial(
    pl.kernel,
    out_type=jax.ShapeDtypeStruct((B, L, 2 * HALF), jnp.float32),
    mesh=_mesh,
    compiler_params=pltpu.CompilerParams(use_tc_tiling_on_sc=True),
    scratch_types=[
        pltpu.VMEM_SHARED((MAX, HALF), jnp.float32),
        pltpu.VMEM_SHARED((MAX, HALF), jnp.float32),
        pltpu.VMEM((NCK, CHL), jnp.int32),
        pltpu.VMEM((NCK, CHL), jnp.int32),
    ] + [pltpu.VMEM((CHL, HALF), jnp.float32)] * (2 * NBUF)
      + [pltpu.SemaphoreType.DMA] * (2 * NBUF),
)
def _emb_lookup(x_hbm, y_hbm, col_hbm, row_hbm, out_hbm,
                colsh, rowsh, xi, yi, *bufsems):
    bufs = bufsems[:2 * NBUF]
    sems = bufsems[2 * NBUF:]
    slots = [(bufs[2 * p], bufs[2 * p + 1], sems[2 * p], sems[2 * p + 1])
             for p in range(NBUF)]

    sid = lax.axis_index("s")
    wid = sid * NC + lax.axis_index("c")
    base_c = wid * NCK

    row0 = sid * STRIPE

    @pl.when(sid < NS - 1)
    def _stage_full():
        pltpu.sync_copy(col_hbm.at[pl.ds(row0, STRIPE)],
                        colsh.at[pl.ds(row0, STRIPE)])
        pltpu.sync_copy(row_hbm.at[pl.ds(row0, STRIPE)],
                        rowsh.at[pl.ds(row0, STRIPE)])

    @pl.when(sid == NS - 1)
    def _stage_tail():
        tail = MAX - (NS - 1) * STRIPE
        t0 = (NS - 1) * STRIPE
        pltpu.sync_copy(col_hbm.at[pl.ds(t0, tail)],
                        colsh.at[pl.ds(t0, tail)])
        pltpu.sync_copy(row_hbm.at[pl.ds(t0, tail)],
                        rowsh.at[pl.ds(t0, tail)])

    pltpu.sync_copy(x_hbm.at[pl.ds(base_c, NCK)], xi)
    pltpu.sync_copy(y_hbm.at[pl.ds(base_c, NCK)], yi)
    plsc.subcore_barrier()

    def issue_gather(k, xb, yb, gsem):
        pltpu.async_copy(colsh.at[xi.at[k]], xb, gsem)
        pltpu.async_copy(rowsh.at[yi.at[k]], yb, gsem)

    def drain_gather(xb, yb, gsem):
        pltpu.make_async_copy(
            xb, out_hbm.at[pl.ds(0, CH), pl.ds(0, L), pl.ds(0, HALF)],
            gsem).wait()
        pltpu.make_async_copy(
            yb, out_hbm.at[pl.ds(0, CH), pl.ds(0, L), pl.ds(HALF, HALF)],
            gsem).wait()

    def issue_write(k, xb, yb, wsem):
        bb = (base_c + k) * CH
        for c in range(CH):
            pltpu.async_copy(
                xb.at[pl.ds(c * L, L)],
                out_hbm.at[bb + c, pl.ds(0, L), pl.ds(0, HALF)], wsem)
            pltpu.async_copy(
                yb.at[pl.ds(c * L, L)],
                out_hbm.at[bb + c, pl.ds(0, L), pl.ds(HALF, HALF)], wsem)

    def drain_write(xb, yb, wsem):
        for c in range(CH):
            pltpu.make_async_copy(
                xb.at[pl.ds(c * L, L)],
                out_hbm.at[0, pl.ds(0, L), pl.ds(0, HALF)], wsem).wait()
            pltpu.make_async_copy(
                yb.at[pl.ds(c * L, L)],
                out_hbm.at[0, pl.ds(0, L), pl.ds(HALF, HALF)], wsem).wait()

    for p, (xb, yb, gsem, wsem) in enumerate(slots):
        issue_gather(p, xb, yb, gsem)

    def ring_body(g, carry):
        k0 = NBUF * g
        for p, (xb, yb, gsem, wsem) in enumerate(slots):
            k = k0 + p
            drain_gather(xb, yb, gsem)
            issue_write(k, xb, yb, wsem)
            drain_write(xb, yb, wsem)
            issue_gather(k + NBUF, xb, yb, gsem)
        return carry

    lax.fori_loop(0, NCK // NBUF - 1, ring_body, 0)

    for p, (xb, yb, gsem, wsem) in enumerate(slots):
        k = NCK - NBUF + p
        drain_gather(xb, yb, gsem)
        issue_write(k, xb, yb, wsem)
        drain_write(xb, yb, wsem)


def kernel(x, y, col_weight, row_weight):
    xr = x.astype(jnp.int32).reshape(B * L // CHL, CHL)
    yr = y.astype(jnp.int32).reshape(B * L // CHL, CHL)
    out = _emb_lookup(xr, yr, col_weight, row_weight)
    return out.reshape(1, B, L, 2 * HALF)


# speedup vs baseline: 1.0412x; 1.0013x over previous
import functools

import jax
import jax.numpy as jnp
from jax import lax
from jax.experimental import pallas as pl
from jax.experimental.pallas import tpu as pltpu
from jax.experimental.pallas import tpu_sc as plsc

MAX = 2000
HALF = 128
B, L = 4096, 50
NC, NS = 2, 16
NW = NC * NS
BB = B // NW
CH = 1
CHL = CH * L
NCK = BB // CH
NBUF = 4
STRIPE = 128

_mesh = plsc.VectorSubcoreMesh(core_axis_name="c", subcore_axis_name="s")


@functools.partial(
    pl.kernel,
    out_type=jax.ShapeDtypeStruct((B, L, 2 * HALF), jnp.float32),
    mesh=_mesh,
    compiler_params=pltpu.CompilerParams(use_tc_tiling_on_sc=True),
    scratch_types=[
        pltpu.VMEM_SHARED((MAX, HALF), jnp.float32),
        pltpu.VMEM_SHARED((MAX, HALF), jnp.float32),
        pltpu.VMEM((NCK, CHL), jnp.int32),
        pltpu.VMEM((NCK, CHL), jnp.int32),
    ] + [pltpu.VMEM((CHL, HALF), jnp.float32)] * (2 * NBUF)
      + [pltpu.SemaphoreType.DMA] * (2 * NBUF),
)
def _emb_lookup(x_hbm, y_hbm, col_hbm, row_hbm, out_hbm,
                colsh, rowsh, xi, yi, *bufsems):
    bufs = bufsems[:2 * NBUF]
    sems = bufsems[2 * NBUF:]
    slots = [(bufs[2 * p], bufs[2 * p + 1], sems[2 * p], sems[2 * p + 1])
             for p in range(NBUF)]

    sid = lax.axis_index("s")
    wid = lax.axis_index("c") * NS + sid
    base_c = wid * NCK

    row0 = sid * STRIPE

    @pl.when(sid < NS - 1)
    def _stage_full():
        pltpu.sync_copy(col_hbm.at[pl.ds(row0, STRIPE)],
                        colsh.at[pl.ds(row0, STRIPE)])
        pltpu.sync_copy(row_hbm.at[pl.ds(row0, STRIPE)],
                        rowsh.at[pl.ds(row0, STRIPE)])

    @pl.when(sid == NS - 1)
    def _stage_tail():
        tail = MAX - (NS - 1) * STRIPE
        t0 = (NS - 1) * STRIPE
        pltpu.sync_copy(col_hbm.at[pl.ds(t0, tail)],
                        colsh.at[pl.ds(t0, tail)])
        pltpu.sync_copy(row_hbm.at[pl.ds(t0, tail)],
                        rowsh.at[pl.ds(t0, tail)])

    pltpu.sync_copy(x_hbm.at[pl.ds(base_c, NCK)], xi)
    pltpu.sync_copy(y_hbm.at[pl.ds(base_c, NCK)], yi)
    plsc.subcore_barrier()

    def issue_gather(k, xb, yb, gsem):
        pltpu.async_copy(colsh.at[xi.at[k]], xb, gsem)
        pltpu.async_copy(rowsh.at[yi.at[k]], yb, gsem)

    def drain_gather(xb, yb, gsem):
        pltpu.make_async_copy(
            xb, out_hbm.at[pl.ds(0, CH), pl.ds(0, L), pl.ds(0, HALF)],
            gsem).wait()
        pltpu.make_async_copy(
            yb, out_hbm.at[pl.ds(0, CH), pl.ds(0, L), pl.ds(HALF, HALF)],
            gsem).wait()

    def issue_write(k, xb, yb, wsem):
        bb = (base_c + k) * CH
        for c in range(CH):
            pltpu.async_copy(
                xb.at[pl.ds(c * L, L)],
                out_hbm.at[bb + c, pl.ds(0, L), pl.ds(0, HALF)], wsem)
            pltpu.async_copy(
                yb.at[pl.ds(c * L, L)],
                out_hbm.at[bb + c, pl.ds(0, L), pl.ds(HALF, HALF)], wsem)

    def drain_write(xb, yb, wsem):
        for c in range(CH):
            pltpu.make_async_copy(
                xb.at[pl.ds(c * L, L)],
                out_hbm.at[0, pl.ds(0, L), pl.ds(0, HALF)], wsem).wait()
            pltpu.make_async_copy(
                yb.at[pl.ds(c * L, L)],
                out_hbm.at[0, pl.ds(0, L), pl.ds(HALF, HALF)], wsem).wait()

    for p, (xb, yb, gsem, wsem) in enumerate(slots):
        issue_gather(p, xb, yb, gsem)

    def ring_body(g, carry):
        k0 = NBUF * g
        for p, (xb, yb, gsem, wsem) in enumerate(slots):
            k = k0 + p
            drain_gather(xb, yb, gsem)
            issue_write(k, xb, yb, wsem)
            drain_write(xb, yb, wsem)
            issue_gather(k + NBUF, xb, yb, gsem)
        return carry

    lax.fori_loop(0, NCK // NBUF - 1, ring_body, 0)

    for p, (xb, yb, gsem, wsem) in enumerate(slots):
        k = NCK - NBUF + p
        drain_gather(xb, yb, gsem)
        issue_write(k, xb, yb, wsem)
        drain_write(xb, yb, wsem)


def kernel(x, y, col_weight, row_weight):
    xr = x.astype(jnp.int32).reshape(B * L // CHL, CHL)
    yr = y.astype(jnp.int32).reshape(B * L // CHL, CHL)
    out = _emb_lookup(xr, yr, col_weight, row_weight)
    return out.reshape(1, B, L, 2 * HALF)
